# baseline (speedup 1.0000x reference)
import jax
import jax.numpy as jnp
from jax import lax
from jax.experimental import pallas as pl
from jax.experimental.pallas import tpu as pltpu, tpu_sc as plsc

B, H, W = 4, 512, 512
HW = H * W
P = B * HW
NC, NS, L = 2, 16, 16
PIX_PER_TILE = HW // NS
CHUNK = 4096
NCHUNK = PIX_PER_TILE // CHUNK
NPAIR = 4 * CHUNK
WIN = 16
HALF = WIN // 2
TBL = WIN * WIN
NTBL = 8
TWORDS = TBL * L
SINK0 = HW
SINKW = 2 * CHUNK
ACC_WORDS = HW + NS * SINKW


def _coords(gy, gx):
    ty = gy.astype(jnp.int32)
    tx = gx.astype(jnp.int32)
    tyf = ty.astype(jnp.float32)
    txf = tx.astype(jnp.float32)
    cy = tyf > gy
    cx = txf > gx
    y0i = jnp.where(cy, ty - 1, ty)
    x0i = jnp.where(cx, tx - 1, tx)
    wy = gy - y0i.astype(jnp.float32)
    wx = gx - x0i.astype(jnp.float32)
    y0 = y0i & (H - 1)
    x0 = x0i & (W - 1)
    y1 = (y0i + 1) & (H - 1)
    x1 = (x0i + 1) & (W - 1)
    w00 = (1.0 - wy) * (1.0 - wx)
    w01 = (1.0 - wy) * wx
    w10 = wy * (1.0 - wx)
    w11 = wy * wx
    yy0 = (y0 + HALF) & (H - 1)
    yy1 = (y1 + HALF) & (H - 1)
    xx0 = (x0 + HALF) & (W - 1)
    xx1 = (x1 + HALF) & (W - 1)
    orv = yy0 | yy1 | xx0 | xx1
    return (y0, x0, y1, x1, w00, w01, w10, w11, yy0, yy1, xx0, xx1, orv)


def _splat_body(phi_hbm, out_hbm,
                gy0, gy1, gx0, gx1, idx_buf, w_buf,
                tb0, tb1, tb2, tb3, tb4, tb5, tb6, tb7, midx,
                acc, sem_in0, sem_in1, sem_z):
    c = lax.axis_index("c")
    s = lax.axis_index("s")
    q = s * PIX_PER_TILE
    sink = SINK0 + s * SINKW
    lane = lax.iota(jnp.int32, L)
    zeros16 = jnp.zeros((L,), jnp.float32)
    izeros16 = jnp.zeros((L,), jnp.int32)
    laneoff = lane << 8

    gy_bufs = (gy0, gy1)
    gx_bufs = (gx0, gx1)
    sem_ins = (sem_in0, sem_in1)
    tbs = (tb0, tb1, tb2, tb3, tb4, tb5, tb6, tb7)

    @pl.loop(0, TBL // L)
    def _init_tbl(j):
        wcell = j * L + lane
        r = wcell >> 4
        col = wcell & (WIN - 1)
        gy_ = (r - HALF) & (H - 1)
        gx_ = (col - HALF) & (W - 1)
        midx[pl.ds(j * L, L)] = (gy_ << 9) + gx_

    ztile = pl.multiple_of(s * PIX_PER_TILE, PIX_PER_TILE)

    for phase in range(2):
        b = 2 * c + phase
        gy_off = pl.multiple_of(b * (2 * HW) + q, CHUNK)
        gx_off = pl.multiple_of(b * (2 * HW) + HW + q, CHUNK)

        for tb in tbs:
            @pl.loop(0, TWORDS // L)
            def _ztbl(j):
                tb[pl.ds(j * L, L)] = zeros16

        @pl.loop(0, PIX_PER_TILE // L)
        def _zero(i):
            w_buf[pl.ds(i * L, L)] = zeros16

        z0 = pltpu.async_copy(w_buf, acc.at[pl.ds(ztile, PIX_PER_TILE)], sem_z)
        z0.wait()
        plsc.subcore_barrier()

        def start_inputs(ch):
            d = ch & 1
            a = pltpu.async_copy(
                phi_hbm.at[pl.ds(gy_off + ch * CHUNK, CHUNK)],
                gy_bufs[d], sem_ins[d])
            bcp = pltpu.async_copy(
                phi_hbm.at[pl.ds(gx_off + ch * CHUNK, CHUNK)],
                gx_bufs[d], sem_ins[d])
            return (a, bcp)

        in_pend = {0: start_inputs(0), 1: start_inputs(1)}

        for ch in range(NCHUNK):
            d = ch & 1
            for cp in in_pend.pop(ch):
                cp.wait()
            gy_buf, gx_buf = gy_bufs[d], gx_bufs[d]

            @pl.loop(0, CHUNK // L, step=NTBL, init_carry=izeros16)
            def _pass1(i0, orall):
                for j in range(NTBL):
                    i = i0 + j
                    tb = tbs[j]
                    gy = gy_buf[pl.ds(i * L, L)]
                    gx = gx_buf[pl.ds(i * L, L)]
                    (y0, x0, y1, x1, w00, w01, w10, w11,
                     yy0, yy1, xx0, xx1, orv) = _coords(gy, gx)
                    m = orv < WIN
                    ry0 = yy0 << 4
                    ry1 = yy1 << 4
                    l00 = laneoff + ((ry0 + xx0) & (TBL - 1))
                    l01 = laneoff + ((ry0 + xx1) & (TBL - 1))
                    l10 = laneoff + ((ry1 + xx0) & (TBL - 1))
                    l11 = laneoff + ((ry1 + xx1) & (TBL - 1))
                    fz = jnp.float32(0.0)
                    cur00 = plsc.load_gather(tb, [l00])
                    cur01 = plsc.load_gather(tb, [l01])
                    cur10 = plsc.load_gather(tb, [l10])
                    cur11 = plsc.load_gather(tb, [l11])
                    plsc.store_scatter(tb, [l00], cur00 + jnp.where(m, w00, fz))
                    plsc.store_scatter(tb, [l01], cur01 + jnp.where(m, w01, fz))
                    plsc.store_scatter(tb, [l10], cur10 + jnp.where(m, w10, fz))
                    plsc.store_scatter(tb, [l11], cur11 + jnp.where(m, w11, fz))
                    orall = orall | orv
                return orall

            any_out = jnp.max(_pass1) >= WIN

            @pl.when(any_out)
            def _pass2():
                @pl.loop(0, CHUNK // L)
                def _stage(i):
                    gy = gy_buf[pl.ds(i * L, L)]
                    gx = gx_buf[pl.ds(i * L, L)]
                    (y0, x0, y1, x1, w00, w01, w10, w11,
                     yy0, yy1, xx0, xx1, orv) = _coords(gy, gx)
                    m = orv < WIN
                    r0 = y0 << 9
                    r1 = y1 << 9
                    o = i * L
                    p0 = sink + o + lane
                    idx_buf[pl.ds(o, L)] = jnp.where(m, p0, r0 + x0)
                    idx_buf[pl.ds(CHUNK + o, L)] = jnp.where(
                        m, p0 + CHUNK, r0 + x1)
                    idx_buf[pl.ds(2 * CHUNK + o, L)] = jnp.where(
                        m, p0, r1 + x0)
                    idx_buf[pl.ds(3 * CHUNK + o, L)] = jnp.where(
                        m, p0 + CHUNK, r1 + x1)
                    w_buf[pl.ds(o, L)] = w00
                    w_buf[pl.ds(CHUNK + o, L)] = w01
                    w_buf[pl.ds(2 * CHUNK + o, L)] = w10
                    w_buf[pl.ds(3 * CHUNK + o, L)] = w11

                pltpu.sync_copy(w_buf, acc.at[idx_buf], add=True)

            if ch + 2 < NCHUNK:
                in_pend[ch + 2] = start_inputs(ch + 2)

        @pl.loop(0, TBL // L)
        def _merge(j):
            v = tb0[pl.ds(TBL + j * L, L)]
            for k in range(2, L):
                v = v + tb0[pl.ds(k * TBL + j * L, L)]
            for tb in tbs[1:]:
                for k in range(L):
                    v = v + tb[pl.ds(k * TBL + j * L, L)]
            tb0[pl.ds(j * L, L)] = tb0[pl.ds(j * L, L)] + v

        pltpu.sync_copy(tb0.at[pl.ds(0, TBL)], acc.at[midx], add=True)
        plsc.subcore_barrier()

        pltpu.sync_copy(
            acc.at[pl.ds(ztile, PIX_PER_TILE)],
            out_hbm.at[pl.ds(pl.multiple_of(b * HW + q, PIX_PER_TILE),
                             PIX_PER_TILE)],
        )


def _make_splat():
    mesh = plsc.VectorSubcoreMesh(core_axis_name="c", subcore_axis_name="s")
    return pl.kernel(
        _splat_body,
        out_type=jax.ShapeDtypeStruct((P,), jnp.float32),
        mesh=mesh,
        compiler_params=pltpu.CompilerParams(needs_layout_passes=False),
        scratch_types=[
            pltpu.VMEM((CHUNK,), jnp.float32),
            pltpu.VMEM((CHUNK,), jnp.float32),
            pltpu.VMEM((CHUNK,), jnp.float32),
            pltpu.VMEM((CHUNK,), jnp.float32),
            pltpu.VMEM((NPAIR,), jnp.int32),
            pltpu.VMEM((NPAIR,), jnp.float32),
            pltpu.VMEM((TWORDS,), jnp.float32),
            pltpu.VMEM((TWORDS,), jnp.float32),
            pltpu.VMEM((TWORDS,), jnp.float32),
            pltpu.VMEM((TWORDS,), jnp.float32),
            pltpu.VMEM((TWORDS,), jnp.float32),
            pltpu.VMEM((TWORDS,), jnp.float32),
            pltpu.VMEM((TWORDS,), jnp.float32),
            pltpu.VMEM((TWORDS,), jnp.float32),
            pltpu.VMEM((TBL,), jnp.int32),
            pltpu.VMEM_SHARED((ACC_WORDS,), jnp.float32),
            pltpu.SemaphoreType.DMA,
            pltpu.SemaphoreType.DMA,
            pltpu.SemaphoreType.DMA,
        ],
    )


_splat = _make_splat()


@jax.jit
def kernel(x, phi):
    del x
    cnt = _splat(phi.reshape(-1))
    return cnt.reshape(B, 1, H, W)


# speedup vs baseline: 1.1186x; 1.1186x over previous
import jax
import jax.numpy as jnp
from jax import lax
from jax.experimental import pallas as pl
from jax.experimental.pallas import tpu as pltpu, tpu_sc as plsc

B, H, W = 4, 512, 512
HW = H * W
P = B * HW
NC, NS, L = 2, 16, 16
PIX_PER_TILE = HW // NS
CHUNK = 4096
NCHUNK = PIX_PER_TILE // CHUNK
NPAIR = 4 * CHUNK
WIN = 16
HALF = WIN // 2
TBL = WIN * WIN
NTBL = 8
TWORDS = TBL * L
SINK0 = HW
SINKW = 2 * CHUNK
ACC_WORDS = HW + NS * SINKW


def _floorfrac(gy, gx):
    ty = gy.astype(jnp.int32)
    tx = gx.astype(jnp.int32)
    tyf = ty.astype(jnp.float32)
    txf = tx.astype(jnp.float32)
    cy = tyf > gy
    cx = txf > gx
    y0i = jnp.where(cy, ty - 1, ty)
    x0i = jnp.where(cx, tx - 1, tx)
    wy = gy - y0i.astype(jnp.float32)
    wx = gx - x0i.astype(jnp.float32)
    yy0 = (y0i + HALF) & (H - 1)
    yy1 = (y0i + HALF + 1) & (H - 1)
    xx0 = (x0i + HALF) & (W - 1)
    xx1 = (x0i + HALF + 1) & (W - 1)
    orv = yy0 | yy1 | xx0 | xx1
    return (y0i, x0i, wy, wx, yy0, yy1, xx0, xx1, orv)


def _splat_body(phi_hbm, out_hbm,
                gy0, gy1, gx0, gx1, idx_buf, w_buf,
                tb0, tb1, tb2, tb3, tb4, tb5, tb6, tb7, midx,
                acc, sem_in0, sem_in1, sem_z):
    c = lax.axis_index("c")
    s = lax.axis_index("s")
    q = s * PIX_PER_TILE
    sink = SINK0 + s * SINKW
    lane = lax.iota(jnp.int32, L)
    zeros16 = jnp.zeros((L,), jnp.float32)
    izeros16 = jnp.zeros((L,), jnp.int32)

    gy_bufs = (gy0, gy1)
    gx_bufs = (gx0, gx1)
    sem_ins = (sem_in0, sem_in1)
    tbs = (tb0, tb1, tb2, tb3, tb4, tb5, tb6, tb7)

    @pl.loop(0, TBL // L)
    def _init_tbl(j):
        wcell = j * L + lane
        r = wcell >> 4
        col = wcell & (WIN - 1)
        gy_ = (r - HALF) & (H - 1)
        gx_ = (col - HALF) & (W - 1)
        midx[pl.ds(j * L, L)] = (gy_ << 9) + gx_

    ztile = pl.multiple_of(s * PIX_PER_TILE, PIX_PER_TILE)

    @pl.loop(0, PIX_PER_TILE // L)
    def _zw(i):
        w_buf[pl.ds(i * L, L)] = zeros16

    for phase in range(2):
        b = 2 * c + phase
        gy_off = pl.multiple_of(b * (2 * HW) + q, CHUNK)
        gx_off = pl.multiple_of(b * (2 * HW) + HW + q, CHUNK)

        for tb in tbs:
            @pl.loop(0, TWORDS // L)
            def _ztbl(j):
                tb[pl.ds(j * L, L)] = zeros16

        z0 = pltpu.async_copy(w_buf, acc.at[pl.ds(ztile, PIX_PER_TILE)], sem_z)
        z0.wait()
        plsc.subcore_barrier()

        def start_inputs(ch):
            d = ch & 1
            a = pltpu.async_copy(
                phi_hbm.at[pl.ds(gy_off + ch * CHUNK, CHUNK)],
                gy_bufs[d], sem_ins[d])
            bcp = pltpu.async_copy(
                phi_hbm.at[pl.ds(gx_off + ch * CHUNK, CHUNK)],
                gx_bufs[d], sem_ins[d])
            return (a, bcp)

        in_pend = {0: start_inputs(0), 1: start_inputs(1)}

        for ch in range(NCHUNK):
            d = ch & 1
            for cp in in_pend.pop(ch):
                cp.wait()
            gy_buf, gx_buf = gy_bufs[d], gx_bufs[d]

            @pl.loop(0, CHUNK // L, step=NTBL, init_carry=izeros16)
            def _pass1(i0, orall):
                for j in range(NTBL):
                    i = i0 + j
                    tb = tbs[j]
                    gy = gy_buf[pl.ds(i * L, L)]
                    gx = gx_buf[pl.ds(i * L, L)]
                    (y0i, x0i, wy, wx,
                     yy0, yy1, xx0, xx1, orv) = _floorfrac(gy, gx)
                    m = orv < WIN
                    fz = jnp.float32(0.0)
                    uym = jnp.where(m, 1.0 - wy, fz)
                    wym = jnp.where(m, wy, fz)
                    ux = 1.0 - wx
                    sy0 = yy0 << 8
                    sy1 = yy1 << 8
                    sx0 = xx0 << 4
                    sx1 = xx1 << 4
                    l00 = ((sy0 + sx0) & (TWORDS - 1)) | lane
                    l01 = ((sy0 + sx1) & (TWORDS - 1)) | lane
                    l10 = ((sy1 + sx0) & (TWORDS - 1)) | lane
                    l11 = ((sy1 + sx1) & (TWORDS - 1)) | lane
                    cur00 = plsc.load_gather(tb, [l00])
                    cur01 = plsc.load_gather(tb, [l01])
                    cur10 = plsc.load_gather(tb, [l10])
                    cur11 = plsc.load_gather(tb, [l11])
                    plsc.store_scatter(tb, [l00], cur00 + uym * ux)
                    plsc.store_scatter(tb, [l01], cur01 + uym * wx)
                    plsc.store_scatter(tb, [l10], cur10 + wym * ux)
                    plsc.store_scatter(tb, [l11], cur11 + wym * wx)
                    orall = orall | orv
                return orall

            any_out = jnp.max(_pass1) >= WIN

            @pl.when(any_out)
            def _pass2():
                @pl.loop(0, CHUNK // L)
                def _stage(i):
                    gy = gy_buf[pl.ds(i * L, L)]
                    gx = gx_buf[pl.ds(i * L, L)]
                    (y0i, x0i, wy, wx,
                     yy0, yy1, xx0, xx1, orv) = _floorfrac(gy, gx)
                    m = orv < WIN
                    uy = 1.0 - wy
                    ux = 1.0 - wx
                    w00 = uy * ux
                    w01 = uy * wx
                    w10 = wy * ux
                    w11 = wy * wx
                    x0 = x0i & (W - 1)
                    x1 = (x0i + 1) & (W - 1)
                    r0 = (y0i & (H - 1)) << 9
                    r1 = ((y0i + 1) & (H - 1)) << 9
                    o = i * L
                    p0 = sink + o + lane
                    idx_buf[pl.ds(o, L)] = jnp.where(m, p0, r0 + x0)
                    idx_buf[pl.ds(CHUNK + o, L)] = jnp.where(
                        m, p0 + CHUNK, r0 + x1)
                    idx_buf[pl.ds(2 * CHUNK + o, L)] = jnp.where(
                        m, p0, r1 + x0)
                    idx_buf[pl.ds(3 * CHUNK + o, L)] = jnp.where(
                        m, p0 + CHUNK, r1 + x1)
                    w_buf[pl.ds(o, L)] = w00
                    w_buf[pl.ds(CHUNK + o, L)] = w01
                    w_buf[pl.ds(2 * CHUNK + o, L)] = w10
                    w_buf[pl.ds(3 * CHUNK + o, L)] = w11

                pltpu.sync_copy(w_buf, acc.at[idx_buf], add=True)

                @pl.loop(0, PIX_PER_TILE // L)
                def _rezero(i):
                    w_buf[pl.ds(i * L, L)] = zeros16

            if ch + 2 < NCHUNK:
                in_pend[ch + 2] = start_inputs(ch + 2)

        @pl.loop(0, TBL // L)
        def _merge(jb):
            out = zeros16
            for t in range(L):
                base = (jb * L + t) << 4
                v = tb0[pl.ds(base, L)]
                for tb in tbs[1:]:
                    v = v + tb[pl.ds(base, L)]
                out = jnp.where(lane == t, jnp.sum(v), out)
            tb0[pl.ds(jb * L, L)] = out

        pltpu.sync_copy(tb0.at[pl.ds(0, TBL)], acc.at[midx], add=True)
        plsc.subcore_barrier()

        pltpu.sync_copy(
            acc.at[pl.ds(ztile, PIX_PER_TILE)],
            out_hbm.at[pl.ds(pl.multiple_of(b * HW + q, PIX_PER_TILE),
                             PIX_PER_TILE)],
        )


def _make_splat():
    mesh = plsc.VectorSubcoreMesh(core_axis_name="c", subcore_axis_name="s")
    return pl.kernel(
        _splat_body,
        out_type=jax.ShapeDtypeStruct((P,), jnp.float32),
        mesh=mesh,
        compiler_params=pltpu.CompilerParams(needs_layout_passes=False),
        scratch_types=[
            pltpu.VMEM((CHUNK,), jnp.float32),
            pltpu.VMEM((CHUNK,), jnp.float32),
            pltpu.VMEM((CHUNK,), jnp.float32),
            pltpu.VMEM((CHUNK,), jnp.float32),
            pltpu.VMEM((NPAIR,), jnp.int32),
            pltpu.VMEM((NPAIR,), jnp.float32),
            pltpu.VMEM((TWORDS,), jnp.float32),
            pltpu.VMEM((TWORDS,), jnp.float32),
            pltpu.VMEM((TWORDS,), jnp.float32),
            pltpu.VMEM((TWORDS,), jnp.float32),
            pltpu.VMEM((TWORDS,), jnp.float32),
            pltpu.VMEM((TWORDS,), jnp.float32),
            pltpu.VMEM((TWORDS,), jnp.float32),
            pltpu.VMEM((TWORDS,), jnp.float32),
            pltpu.VMEM((TBL,), jnp.int32),
            pltpu.VMEM_SHARED((ACC_WORDS,), jnp.float32),
            pltpu.SemaphoreType.DMA,
            pltpu.SemaphoreType.DMA,
            pltpu.SemaphoreType.DMA,
        ],
    )


_splat = _make_splat()


@jax.jit
def kernel(x, phi):
    del x
    cnt = _splat(phi.reshape(-1))
    return cnt.reshape(B, 1, H, W)
